# baseline (speedup 1.0000x reference)
import functools

import jax
import jax.numpy as jnp
from jax import lax
from jax.experimental import pallas as pl
from jax.experimental.pallas import tpu as pltpu
from jax.experimental.pallas import tpu_sc as plsc

N = 10000
NP = 10240
F = 128
D = 64
CW = 32
H = 8
HD = H * D
E = 320000
ET = E + N
NC = 2
NS = 16
CH = 128
KB = 3
EP = 331776
RPT = NP // NS
PADR = 8

_HIGH = jax.lax.Precision.HIGHEST


def _dot(a, b):
    return jnp.dot(a, b, precision=_HIGH, preferred_element_type=jnp.float32)


def _tc_a_body(x_ref, w1_ref, as_ref, ad_ref, wr1_ref, br1_ref,
               xw1s_ref, a_s1_ref, a_d1_ref, xr1_ref):
    xb = x_ref[...]
    w1 = w1_ref[...]
    a_s = as_ref[...]
    a_d = ad_ref[...]
    for h in range(H):
        xw_h = _dot(xb, w1[:, h * D:(h + 1) * D])
        for c in range(NC):
            xw1s_ref[c, h] = xw_h[:, c * CW:(c + 1) * CW]
        a_s1_ref[h] = jnp.sum(xw_h * a_s[h][None, :], axis=-1)
        a_d1_ref[h] = jnp.sum(xw_h * a_d[h][None, :], axis=-1)
    xr1_ref[...] = _dot(xb, wr1_ref[...]) + br1_ref[...]


def _tc_a(xpad, W1, att_src1, att_dst1, Wr1, br1r):
    B = 512
    G = NP // B
    return pl.pallas_call(
        _tc_a_body,
        grid=(G,),
        in_specs=[
            pl.BlockSpec((B, F), lambda i: (i, 0)),
            pl.BlockSpec((F, HD), lambda i: (0, 0)),
            pl.BlockSpec((H, D), lambda i: (0, 0)),
            pl.BlockSpec((H, D), lambda i: (0, 0)),
            pl.BlockSpec((F, HD), lambda i: (0, 0)),
            pl.BlockSpec((1, HD), lambda i: (0, 0)),
        ],
        out_specs=[
            pl.BlockSpec((NC, H, B, CW), lambda i: (0, 0, i, 0)),
            pl.BlockSpec((H, B), lambda i: (0, i)),
            pl.BlockSpec((H, B), lambda i: (0, i)),
            pl.BlockSpec((B, HD), lambda i: (i, 0)),
        ],
        out_shape=[
            jax.ShapeDtypeStruct((NC, H, NP, CW), jnp.float32),
            jax.ShapeDtypeStruct((H, NP), jnp.float32),
            jax.ShapeDtypeStruct((H, NP), jnp.float32),
            jax.ShapeDtypeStruct((NP, HD), jnp.float32),
        ],
    )(xpad, W1, att_src1, att_dst1, Wr1, br1r)


def _tc_b_body(acc_ref, den_ref, xr1_ref, b1_ref, w2_ref, as2_ref, ad2_ref,
               wr2_ref, br2_ref, xw2s_ref, aux2_ref, xr2_ref):
    w2 = w2_ref[...]
    wr2 = wr2_ref[...]
    b1 = b1_ref[...]
    xw2 = jnp.zeros((xr2_ref.shape[0], D), jnp.float32)
    xr2 = jnp.zeros((xr2_ref.shape[0], D), jnp.float32)
    for h in range(H):
        d = jnp.maximum(den_ref[0, h], 1e-30)[:, None]
        acc_h = jnp.concatenate([acc_ref[0, h], acc_ref[1, h]], axis=1)
        hh = acc_h / d + b1[0, h * D:(h + 1) * D][None, :]
        hh = jnp.where(hh > 0, hh, jnp.exp(jnp.minimum(hh, 0.0)) - 1.0)
        hb = hh + xr1_ref[:, h * D:(h + 1) * D]
        xw2 = xw2 + _dot(hb, w2[h * D:(h + 1) * D, :])
        xr2 = xr2 + _dot(hb, wr2[h * D:(h + 1) * D, :])
    for c in range(NC):
        xw2s_ref[c] = xw2[:, c * CW:(c + 1) * CW]
    aux2_ref[0] = jnp.sum(xw2 * as2_ref[...][0][None, :], axis=-1)
    aux2_ref[1] = jnp.sum(xw2 * ad2_ref[...][0][None, :], axis=-1)
    xr2_ref[...] = xr2 + br2_ref[...]


def _tc_b(acc1, den1, xr1, b1r, W2, att_src2, att_dst2, Wr2, br2r):
    B = 512
    G = NP // B
    return pl.pallas_call(
        _tc_b_body,
        grid=(G,),
        in_specs=[
            pl.BlockSpec((NC, H, B, CW), lambda i: (0, 0, i, 0)),
            pl.BlockSpec((NC, H, B), lambda i: (0, 0, i)),
            pl.BlockSpec((B, HD), lambda i: (i, 0)),
            pl.BlockSpec((1, HD), lambda i: (0, 0)),
            pl.BlockSpec((HD, D), lambda i: (0, 0)),
            pl.BlockSpec((1, D), lambda i: (0, 0)),
            pl.BlockSpec((1, D), lambda i: (0, 0)),
            pl.BlockSpec((HD, D), lambda i: (0, 0)),
            pl.BlockSpec((1, D), lambda i: (0, 0)),
        ],
        out_specs=[
            pl.BlockSpec((NC, B, CW), lambda i: (0, i, 0)),
            pl.BlockSpec((2, B), lambda i: (0, i)),
            pl.BlockSpec((B, D), lambda i: (i, 0)),
        ],
        out_shape=[
            jax.ShapeDtypeStruct((NC, NP, CW), jnp.float32),
            jax.ShapeDtypeStruct((2, NP), jnp.float32),
            jax.ShapeDtypeStruct((NP, D), jnp.float32),
        ],
    )(acc1, den1, xr1, b1r, W2, att_src2, att_dst2, Wr2, br2r)


def _tc_c_body(acc_ref, den_ref, xr2_ref, b2_ref, out_ref):
    a = jnp.concatenate([acc_ref[0], acc_ref[1]], axis=1)
    d = jnp.maximum(den_ref[0], 1e-30)[:, None]
    h2 = a / d + b2_ref[...] + xr2_ref[...]
    m = jnp.max(h2, axis=1, keepdims=True)
    ex = jnp.exp(h2 - m)
    out_ref[...] = h2 - m - jnp.log(jnp.sum(ex, axis=1, keepdims=True))


def _tc_c(acc2, den2, xr2, b2r):
    B = 512
    G = NP // B
    return pl.pallas_call(
        _tc_c_body,
        grid=(G,),
        in_specs=[
            pl.BlockSpec((NC, B, CW), lambda i: (0, i, 0)),
            pl.BlockSpec((NC, B), lambda i: (0, i)),
            pl.BlockSpec((B, D), lambda i: (i, 0)),
            pl.BlockSpec((1, D), lambda i: (0, 0)),
        ],
        out_specs=pl.BlockSpec((B, D), lambda i: (i, 0)),
        out_shape=jax.ShapeDtypeStruct((NP, D), jnp.float32),
    )(acc2, den2, xr2, b2r)


def _zero_fill(zrow, zden):
    z16 = jnp.zeros((16,), jnp.float32)
    for r in range(64):
        for q in range(CW // 16):
            zrow[r, pl.ds(q * 16, 16)] = z16
    for r in range(RPT // 16):
        zden[pl.ds(r * 16, 16)] = z16


def _edge_sweep(nchunks, ebase, spm_tab, srcp, dstp, as_t, ad_t,
                idxs, idxd, exbs, rowbs, spm_acc, spm_den, sgs, sss, sds):
    def it(i, carry):
        c0 = ebase + i * (KB * CH)

        @pl.when(i > 0)
        def _drain():
            for q in range(KB):
                pltpu.make_async_copy(rowbs[q], spm_acc.at[idxd[q]],
                                      sss[q]).wait()
                pltpu.make_async_copy(exbs[q], spm_den.at[idxd[q]],
                                      sds[q]).wait()

        gat = []
        for q in range(KB):
            pltpu.sync_copy(srcp.at[pl.ds(c0 + q * CH, CH)], idxs[q])
            pltpu.sync_copy(dstp.at[pl.ds(c0 + q * CH, CH)], idxd[q])
            gat.append(pltpu.async_copy(spm_tab.at[idxs[q]], rowbs[q],
                                        sgs[q]))
        for q in range(KB):
            exq, idq, isq, rbq = exbs[q], idxd[q], idxs[q], rowbs[q]

            def exbody(j, c, exq=exq, idq=idq, isq=isq):
                sl = pl.ds(j * 16, 16)
                s16 = isq[sl]
                d16 = idq[sl]
                al = (plsc.load_gather(as_t, [s16])
                      + plsc.load_gather(ad_t, [d16]))
                al = jnp.where(al >= 0, al, al * jnp.float32(0.2))
                exq[sl] = jnp.exp(al)
                return c

            lax.fori_loop(0, CH // 16, exbody, 0)
            gat[q].wait()

            def sbody(j, c, exq=exq, rbq=rbq):
                exv = exq[pl.ds(j * 16, 16)]
                for t in range(16):
                    k = j * 16 + t
                    ev = exv[t]
                    for w in range(CW // 16):
                        rbq[k, pl.ds(w * 16, 16)] = (
                            rbq[k, pl.ds(w * 16, 16)] * ev)
                return c

            lax.fori_loop(0, CH // 16, sbody, 0)
            pltpu.async_copy(rowbs[q], spm_acc.at[idxd[q]], sss[q], add=True)
            pltpu.async_copy(exbs[q], spm_den.at[idxd[q]], sds[q], add=True)
        return carry

    lax.fori_loop(0, nchunks // KB, it, 0)
    for q in range(KB):
        pltpu.make_async_copy(rowbs[q], spm_acc.at[idxd[q]], sss[q]).wait()
        pltpu.make_async_copy(exbs[q], spm_den.at[idxd[q]], sds[q]).wait()


def _stage_and_zero(rb, tab_src, as_src, ad_src, as_t, ad_t,
                    spm_tab, spm_acc, spm_den, zrow, zden):
    pltpu.sync_copy(tab_src, spm_tab.at[pl.ds(rb, RPT)])
    for b in range(RPT // 64):
        pltpu.sync_copy(zrow, spm_acc.at[pl.ds(rb + b * 64, 64)])
    pltpu.sync_copy(zden, spm_den.at[pl.ds(rb, RPT)])
    pltpu.sync_copy(as_src, as_t)
    pltpu.sync_copy(ad_src, ad_t)


def _split_bufs(bufs):
    idxs = bufs[0:KB]
    idxd = bufs[KB:2 * KB]
    exbs = bufs[2 * KB:3 * KB]
    rowbs = bufs[3 * KB:4 * KB]
    rest = bufs[4 * KB:]
    zrow, zden = rest[0], rest[1]
    sgs = rest[2:2 + KB]
    sss = rest[2 + KB:2 + 2 * KB]
    sds = rest[2 + 2 * KB:2 + 3 * KB]
    return idxs, idxd, exbs, rowbs, zrow, zden, sgs, sss, sds


@functools.lru_cache(maxsize=None)
def _sc_kernels():
    scratch = (
        [
            pltpu.VMEM_SHARED((NP, CW), jnp.float32),
            pltpu.VMEM_SHARED((NP, CW), jnp.float32),
            pltpu.VMEM_SHARED((NP,), jnp.float32),
            pltpu.VMEM((NP,), jnp.float32),
            pltpu.VMEM((NP,), jnp.float32),
        ]
        + [pltpu.VMEM((CH,), jnp.int32) for _ in range(KB)]
        + [pltpu.VMEM((CH,), jnp.int32) for _ in range(KB)]
        + [pltpu.VMEM((CH,), jnp.float32) for _ in range(KB)]
        + [pltpu.VMEM((CH, CW), jnp.float32) for _ in range(KB)]
        + [
            pltpu.VMEM((64, CW), jnp.float32),
            pltpu.VMEM((RPT,), jnp.float32),
        ]
        + [pltpu.SemaphoreType.DMA for _ in range(3 * KB)]
    )
    mesh = plsc.VectorSubcoreMesh(core_axis_name="c", subcore_axis_name="s",
                                  num_cores=NC, num_subcores=NS)

    @functools.partial(
        pl.kernel,
        out_type=[
            jax.ShapeDtypeStruct((NC, H, NP, CW), jnp.float32),
            jax.ShapeDtypeStruct((NC, H, NP), jnp.float32),
        ],
        mesh=mesh,
        scratch_types=scratch,
        compiler_params=pltpu.CompilerParams(needs_layout_passes=False),
    )
    def sc_l1(xw1s, a_s1, a_d1, srcp, dstp, acc1, den1,
              spm_tab, spm_acc, spm_den, as_t, ad_t, *bufs):
        (idxs, idxd, exbs, rowbs, zrow, zden,
         sgs, sss, sds) = _split_bufs(bufs)
        cid = lax.axis_index("c")
        tile = lax.axis_index("s")
        rb = tile * RPT
        _zero_fill(zrow, zden)
        epw = EP // NS
        nch = epw // CH
        for h in range(H):
            _stage_and_zero(rb, xw1s.at[cid, h, pl.ds(rb, RPT)],
                            a_s1.at[h], a_d1.at[h], as_t, ad_t,
                            spm_tab, spm_acc, spm_den, zrow, zden)
            plsc.subcore_barrier()
            _edge_sweep(nch, tile * epw, spm_tab, srcp, dstp, as_t, ad_t,
                        idxs, idxd, exbs, rowbs, spm_acc, spm_den,
                        sgs, sss, sds)
            plsc.subcore_barrier()
            pltpu.sync_copy(spm_acc.at[pl.ds(rb, RPT)],
                            acc1.at[cid, h, pl.ds(rb, RPT)])
            pltpu.sync_copy(spm_den.at[pl.ds(rb, RPT)],
                            den1.at[cid, h, pl.ds(rb, RPT)])
            plsc.subcore_barrier()

    @functools.partial(
        pl.kernel,
        out_type=[
            jax.ShapeDtypeStruct((NC, NP, CW), jnp.float32),
            jax.ShapeDtypeStruct((NC, NP), jnp.float32),
        ],
        mesh=mesh,
        scratch_types=scratch,
        compiler_params=pltpu.CompilerParams(needs_layout_passes=False),
    )
    def sc_l2(xw2s, aux2, srcp, dstp, acc2, den2,
              spm_tab, spm_acc, spm_den, as_t, ad_t, *bufs):
        (idxs, idxd, exbs, rowbs, zrow, zden,
         sgs, sss, sds) = _split_bufs(bufs)
        cid = lax.axis_index("c")
        tile = lax.axis_index("s")
        rb = tile * RPT
        _zero_fill(zrow, zden)
        _stage_and_zero(rb, xw2s.at[cid, pl.ds(rb, RPT)], aux2.at[0],
                        aux2.at[1], as_t, ad_t, spm_tab, spm_acc, spm_den,
                        zrow, zden)
        plsc.subcore_barrier()
        epw = EP // NS
        _edge_sweep(epw // CH, tile * epw, spm_tab, srcp, dstp, as_t, ad_t,
                    idxs, idxd, exbs, rowbs, spm_acc, spm_den,
                    sgs, sss, sds)
        plsc.subcore_barrier()
        pltpu.sync_copy(spm_acc.at[pl.ds(rb, RPT)],
                        acc2.at[cid, pl.ds(rb, RPT)])
        pltpu.sync_copy(spm_den.at[pl.ds(rb, RPT)],
                        den2.at[cid, pl.ds(rb, RPT)])

    return sc_l1, sc_l2


def kernel(x, edge_index, W1, att_src1, att_dst1, b1, W2, att_src2, att_dst2,
           b2, Wr1, br1, Wr2, br2):
    xpad = jnp.pad(x, ((0, NP - N), (0, 0)))
    loops = jnp.arange(N, dtype=jnp.int32)
    padv = (N + (jnp.arange(EP - ET, dtype=jnp.int32) % PADR)).astype(jnp.int32)
    srcp = jnp.concatenate([edge_index[0].astype(jnp.int32), loops, padv])
    dstp = jnp.concatenate([edge_index[1].astype(jnp.int32), loops, padv])

    sc_l1, sc_l2 = _sc_kernels()
    xw1s, a_s1, a_d1, xr1 = _tc_a(xpad, W1, att_src1, att_dst1, Wr1,
                                  br1.reshape(1, HD))
    acc1, den1 = sc_l1(xw1s, a_s1, a_d1, srcp, dstp)
    xw2s, aux2, xr2 = _tc_b(acc1, den1, xr1, b1.reshape(1, HD), W2,
                            att_src2, att_dst2, Wr2, br2.reshape(1, D))
    acc2, den2 = sc_l2(xw2s, aux2, srcp, dstp)
    out = _tc_c(acc2, den2, xr2, b2.reshape(1, D))
    return out[:N]


# speedup vs baseline: 1.2017x; 1.2017x over previous
import functools

import jax
import jax.numpy as jnp
from jax import lax
from jax.experimental import pallas as pl
from jax.experimental.pallas import tpu as pltpu
from jax.experimental.pallas import tpu_sc as plsc

N = 10000
NP = 10240
F = 128
D = 64
H = 8
HD = H * D
E = 320000
ET = E + N
NC = 2
NS = 16
CH = 128
EP = 331776
RPT = NP // NS
PADR = 8

_HIGH = jax.lax.Precision.HIGHEST


def _dot(a, b):
    return jnp.dot(a, b, precision=_HIGH, preferred_element_type=jnp.float32)


def _tc_a_body(x_ref, w1_ref, as_ref, ad_ref, wr1_ref, br1_ref,
               xw1h_ref, a_s1_ref, a_d1_ref, xr1_ref):
    xb = x_ref[...]
    w1 = w1_ref[...]
    a_s = as_ref[...]
    a_d = ad_ref[...]
    for h in range(H):
        xw_h = _dot(xb, w1[:, h * D:(h + 1) * D])
        xw1h_ref[h] = xw_h
        a_s1_ref[h] = jnp.sum(xw_h * a_s[h][None, :], axis=-1)
        a_d1_ref[h] = jnp.sum(xw_h * a_d[h][None, :], axis=-1)
    xr1_ref[...] = _dot(xb, wr1_ref[...]) + br1_ref[...]


def _tc_a(xpad, W1, att_src1, att_dst1, Wr1, br1r):
    B = 512
    G = NP // B
    return pl.pallas_call(
        _tc_a_body,
        grid=(G,),
        in_specs=[
            pl.BlockSpec((B, F), lambda i: (i, 0)),
            pl.BlockSpec((F, HD), lambda i: (0, 0)),
            pl.BlockSpec((H, D), lambda i: (0, 0)),
            pl.BlockSpec((H, D), lambda i: (0, 0)),
            pl.BlockSpec((F, HD), lambda i: (0, 0)),
            pl.BlockSpec((1, HD), lambda i: (0, 0)),
        ],
        out_specs=[
            pl.BlockSpec((H, B, D), lambda i: (0, i, 0)),
            pl.BlockSpec((H, B), lambda i: (0, i)),
            pl.BlockSpec((H, B), lambda i: (0, i)),
            pl.BlockSpec((B, HD), lambda i: (i, 0)),
        ],
        out_shape=[
            jax.ShapeDtypeStruct((H, NP, D), jnp.float32),
            jax.ShapeDtypeStruct((H, NP), jnp.float32),
            jax.ShapeDtypeStruct((H, NP), jnp.float32),
            jax.ShapeDtypeStruct((NP, HD), jnp.float32),
        ],
    )(xpad, W1, att_src1, att_dst1, Wr1, br1r)


def _tc_b_body(acc_ref, den_ref, xr1_ref, b1_ref, w2_ref, as2_ref, ad2_ref,
               wr2_ref, br2_ref, xw2_ref, aux2_ref, xr2_ref):
    w2 = w2_ref[...]
    wr2 = wr2_ref[...]
    b1 = b1_ref[...]
    xw2 = jnp.zeros(xw2_ref.shape, jnp.float32)
    xr2 = jnp.zeros(xr2_ref.shape, jnp.float32)
    for h in range(H):
        d = jnp.maximum(den_ref[h], 1e-30)[:, None]
        hh = acc_ref[h] / d + b1[0, h * D:(h + 1) * D][None, :]
        hh = jnp.where(hh > 0, hh, jnp.exp(jnp.minimum(hh, 0.0)) - 1.0)
        hb = hh + xr1_ref[:, h * D:(h + 1) * D]
        xw2 = xw2 + _dot(hb, w2[h * D:(h + 1) * D, :])
        xr2 = xr2 + _dot(hb, wr2[h * D:(h + 1) * D, :])
    xw2_ref[...] = xw2
    aux2_ref[0] = jnp.sum(xw2 * as2_ref[...][0][None, :], axis=-1)
    aux2_ref[1] = jnp.sum(xw2 * ad2_ref[...][0][None, :], axis=-1)
    xr2_ref[...] = xr2 + br2_ref[...]


def _tc_b(acc1, den1, xr1, b1r, W2, att_src2, att_dst2, Wr2, br2r):
    B = 512
    G = NP // B
    return pl.pallas_call(
        _tc_b_body,
        grid=(G,),
        in_specs=[
            pl.BlockSpec((H, B, D), lambda i: (0, i, 0)),
            pl.BlockSpec((H, B), lambda i: (0, i)),
            pl.BlockSpec((B, HD), lambda i: (i, 0)),
            pl.BlockSpec((1, HD), lambda i: (0, 0)),
            pl.BlockSpec((HD, D), lambda i: (0, 0)),
            pl.BlockSpec((1, D), lambda i: (0, 0)),
            pl.BlockSpec((1, D), lambda i: (0, 0)),
            pl.BlockSpec((HD, D), lambda i: (0, 0)),
            pl.BlockSpec((1, D), lambda i: (0, 0)),
        ],
        out_specs=[
            pl.BlockSpec((B, D), lambda i: (i, 0)),
            pl.BlockSpec((2, B), lambda i: (0, i)),
            pl.BlockSpec((B, D), lambda i: (i, 0)),
        ],
        out_shape=[
            jax.ShapeDtypeStruct((NP, D), jnp.float32),
            jax.ShapeDtypeStruct((2, NP), jnp.float32),
            jax.ShapeDtypeStruct((NP, D), jnp.float32),
        ],
    )(acc1, den1, xr1, b1r, W2, att_src2, att_dst2, Wr2, br2r)


def _tc_c_body(acc_ref, den_ref, xr2_ref, b2_ref, out_ref):
    a = acc_ref[0] + acc_ref[1]
    d = jnp.maximum(den_ref[0] + den_ref[1], 1e-30)[:, None]
    h2 = a / d + b2_ref[...] + xr2_ref[...]
    m = jnp.max(h2, axis=1, keepdims=True)
    ex = jnp.exp(h2 - m)
    out_ref[...] = h2 - m - jnp.log(jnp.sum(ex, axis=1, keepdims=True))


def _tc_c(acc2, den2, xr2, b2r):
    B = 512
    G = NP // B
    return pl.pallas_call(
        _tc_c_body,
        grid=(G,),
        in_specs=[
            pl.BlockSpec((2, B, D), lambda i: (0, i, 0)),
            pl.BlockSpec((2, B), lambda i: (0, i)),
            pl.BlockSpec((B, D), lambda i: (i, 0)),
            pl.BlockSpec((1, D), lambda i: (0, 0)),
        ],
        out_specs=pl.BlockSpec((B, D), lambda i: (i, 0)),
        out_shape=jax.ShapeDtypeStruct((NP, D), jnp.float32),
    )(acc2, den2, xr2, b2r)


def _zero_fill(zrow, zden):
    z16 = jnp.zeros((16,), jnp.float32)
    for r in range(64):
        for q in range(D // 16):
            zrow[r, pl.ds(q * 16, 16)] = z16
    for r in range(RPT // 16):
        zden[pl.ds(r * 16, 16)] = z16


def _edge_sweep(nchunks, ebase, srcp, dstp, as_t, ad_t, idx_s, idx_d, exb,
                rowb, spm_tab, spm_acc, spm_den, sem0, sem1, sem2):
    def chunk(c, carry):
        base = ebase + c * CH
        pltpu.sync_copy(srcp.at[pl.ds(base, CH)], idx_s)
        pltpu.sync_copy(dstp.at[pl.ds(base, CH)], idx_d)
        for j in range(CH // 16):
            sl = pl.ds(j * 16, 16)
            s16 = idx_s[sl]
            d16 = idx_d[sl]
            al = plsc.load_gather(as_t, [s16]) + plsc.load_gather(ad_t, [d16])
            al = jnp.where(al >= 0, al, al * jnp.float32(0.2))
            exb[sl] = jnp.exp(al)
        pltpu.async_copy(spm_tab.at[idx_s], rowb, sem0).wait()
        for j in range(CH // 16):
            exv = exb[pl.ds(j * 16, 16)]
            for t in range(16):
                k = j * 16 + t
                ev = exv[t]
                for q in range(D // 16):
                    rowb[k, pl.ds(q * 16, 16)] = (
                        rowb[k, pl.ds(q * 16, 16)] * ev)
        pltpu.async_copy(rowb, spm_acc.at[idx_d], sem1, add=True).wait()
        pltpu.async_copy(exb, spm_den.at[idx_d], sem2, add=True).wait()
        return carry

    lax.fori_loop(0, nchunks, chunk, 0)


def _stage_and_zero(rb, tab_src, as_src, ad_src, as_t, ad_t,
                    spm_tab, spm_acc, spm_den, zrow, zden):
    pltpu.sync_copy(tab_src, spm_tab.at[pl.ds(rb, RPT)])
    for b in range(RPT // 64):
        pltpu.sync_copy(zrow, spm_acc.at[pl.ds(rb + b * 64, 64)])
    pltpu.sync_copy(zden, spm_den.at[pl.ds(rb, RPT)])
    pltpu.sync_copy(as_src, as_t)
    pltpu.sync_copy(ad_src, ad_t)


@functools.lru_cache(maxsize=None)
def _sc_kernels():
    scratch = [
        pltpu.VMEM_SHARED((NP, D), jnp.float32),
        pltpu.VMEM_SHARED((NP, D), jnp.float32),
        pltpu.VMEM_SHARED((NP,), jnp.float32),
        pltpu.VMEM((NP,), jnp.float32),
        pltpu.VMEM((NP,), jnp.float32),
        pltpu.VMEM((CH,), jnp.int32),
        pltpu.VMEM((CH,), jnp.int32),
        pltpu.VMEM((CH,), jnp.float32),
        pltpu.VMEM((CH, D), jnp.float32),
        pltpu.VMEM((64, D), jnp.float32),
        pltpu.VMEM((RPT,), jnp.float32),
        pltpu.SemaphoreType.DMA,
        pltpu.SemaphoreType.DMA,
        pltpu.SemaphoreType.DMA,
    ]
    mesh = plsc.VectorSubcoreMesh(core_axis_name="c", subcore_axis_name="s",
                                  num_cores=NC, num_subcores=NS)

    @functools.partial(
        pl.kernel,
        out_type=[
            jax.ShapeDtypeStruct((H, NP, D), jnp.float32),
            jax.ShapeDtypeStruct((H, NP), jnp.float32),
        ],
        mesh=mesh,
        scratch_types=scratch,
        compiler_params=pltpu.CompilerParams(needs_layout_passes=False),
    )
    def sc_l1(xw1h, a_s1, a_d1, srcp, dstp, acc1, den1,
              spm_tab, spm_acc, spm_den, as_t, ad_t, idx_s, idx_d, exb, rowb,
              zrow, zden, sem0, sem1, sem2):
        cid = lax.axis_index("c")
        tile = lax.axis_index("s")
        rb = tile * RPT
        _zero_fill(zrow, zden)
        epw = EP // NS
        nch = epw // CH
        for hi in range(H // NC):
            h = cid * (H // NC) + hi
            _stage_and_zero(rb, xw1h.at[h, pl.ds(rb, RPT)], a_s1.at[h],
                            a_d1.at[h], as_t, ad_t, spm_tab, spm_acc,
                            spm_den, zrow, zden)
            plsc.subcore_barrier()
            _edge_sweep(nch, tile * epw, srcp, dstp, as_t, ad_t, idx_s,
                        idx_d, exb, rowb, spm_tab, spm_acc, spm_den,
                        sem0, sem1, sem2)
            plsc.subcore_barrier()
            pltpu.sync_copy(spm_acc.at[pl.ds(rb, RPT)],
                            acc1.at[h, pl.ds(rb, RPT)])
            pltpu.sync_copy(spm_den.at[pl.ds(rb, RPT)],
                            den1.at[h, pl.ds(rb, RPT)])
            plsc.subcore_barrier()

    @functools.partial(
        pl.kernel,
        out_type=[
            jax.ShapeDtypeStruct((NC, NP, D), jnp.float32),
            jax.ShapeDtypeStruct((NC, NP), jnp.float32),
        ],
        mesh=mesh,
        scratch_types=scratch,
        compiler_params=pltpu.CompilerParams(needs_layout_passes=False),
    )
    def sc_l2(xw2, aux2, srcp, dstp, acc2, den2,
              spm_tab, spm_acc, spm_den, as_t, ad_t, idx_s, idx_d, exb, rowb,
              zrow, zden, sem0, sem1, sem2):
        cid = lax.axis_index("c")
        tile = lax.axis_index("s")
        rb = tile * RPT
        _zero_fill(zrow, zden)
        _stage_and_zero(rb, xw2.at[pl.ds(rb, RPT)], aux2.at[0], aux2.at[1],
                        as_t, ad_t, spm_tab, spm_acc, spm_den, zrow, zden)
        plsc.subcore_barrier()
        epc = EP // NC
        epw = epc // NS
        _edge_sweep(epw // CH, cid * epc + tile * epw, srcp, dstp, as_t,
                    ad_t, idx_s, idx_d, exb, rowb, spm_tab, spm_acc,
                    spm_den, sem0, sem1, sem2)
        plsc.subcore_barrier()
        pltpu.sync_copy(spm_acc.at[pl.ds(rb, RPT)],
                        acc2.at[cid, pl.ds(rb, RPT)])
        pltpu.sync_copy(spm_den.at[pl.ds(rb, RPT)],
                        den2.at[cid, pl.ds(rb, RPT)])

    return sc_l1, sc_l2


def kernel(x, edge_index, W1, att_src1, att_dst1, b1, W2, att_src2, att_dst2,
           b2, Wr1, br1, Wr2, br2):
    xpad = jnp.pad(x, ((0, NP - N), (0, 0)))
    loops = jnp.arange(N, dtype=jnp.int32)
    padv = (N + (jnp.arange(EP - ET, dtype=jnp.int32) % PADR)).astype(jnp.int32)
    srcp = jnp.concatenate([edge_index[0].astype(jnp.int32), loops, padv])
    dstp = jnp.concatenate([edge_index[1].astype(jnp.int32), loops, padv])

    sc_l1, sc_l2 = _sc_kernels()
    xw1h, a_s1, a_d1, xr1 = _tc_a(xpad, W1, att_src1, att_dst1, Wr1,
                                  br1.reshape(1, HD))
    acc1, den1 = sc_l1(xw1h, a_s1, a_d1, srcp, dstp)
    xw2, aux2, xr2 = _tc_b(acc1, den1, xr1, b1.reshape(1, HD), W2,
                           att_src2, att_dst2, Wr2, br2.reshape(1, D))
    acc2, den2 = sc_l2(xw2, aux2, srcp, dstp)
    out = _tc_c(acc2, den2, xr2, b2.reshape(1, D))
    return out[:N]


# speedup vs baseline: 1.2470x; 1.0377x over previous
import functools

import jax
import jax.numpy as jnp
from jax import lax
from jax.experimental import pallas as pl
from jax.experimental.pallas import tpu as pltpu
from jax.experimental.pallas import tpu_sc as plsc

N = 10000
NP = 10240
F = 128
D = 64
H = 8
HD = H * D
E = 320000
ET = E + N
NC = 2
NS = 16
CH = 128
EP = 331776
RPT = NP // NS
PADR = 8

_HIGH = jax.lax.Precision.HIGHEST


def _dot(a, b):
    return jnp.dot(a, b, precision=_HIGH, preferred_element_type=jnp.float32)


def _tc_a_body(x_ref, w1_ref, as_ref, ad_ref, wr1_ref, br1_ref,
               xw1h_ref, a_s1_ref, a_d1_ref, xr1_ref):
    xb = x_ref[...]
    w1 = w1_ref[...]
    a_s = as_ref[...]
    a_d = ad_ref[...]
    for h in range(H):
        xw_h = _dot(xb, w1[:, h * D:(h + 1) * D])
        xw1h_ref[h] = xw_h
        a_s1_ref[h] = jnp.sum(xw_h * a_s[h][None, :], axis=-1)
        a_d1_ref[h] = jnp.sum(xw_h * a_d[h][None, :], axis=-1)
    xr1_ref[...] = _dot(xb, wr1_ref[...]) + br1_ref[...]


def _tc_a(xpad, W1, att_src1, att_dst1, Wr1, br1r):
    B = 512
    G = NP // B
    return pl.pallas_call(
        _tc_a_body,
        grid=(G,),
        in_specs=[
            pl.BlockSpec((B, F), lambda i: (i, 0)),
            pl.BlockSpec((F, HD), lambda i: (0, 0)),
            pl.BlockSpec((H, D), lambda i: (0, 0)),
            pl.BlockSpec((H, D), lambda i: (0, 0)),
            pl.BlockSpec((F, HD), lambda i: (0, 0)),
            pl.BlockSpec((1, HD), lambda i: (0, 0)),
        ],
        out_specs=[
            pl.BlockSpec((H, B, D), lambda i: (0, i, 0)),
            pl.BlockSpec((H, B), lambda i: (0, i)),
            pl.BlockSpec((H, B), lambda i: (0, i)),
            pl.BlockSpec((B, HD), lambda i: (i, 0)),
        ],
        out_shape=[
            jax.ShapeDtypeStruct((H, NP, D), jnp.float32),
            jax.ShapeDtypeStruct((H, NP), jnp.float32),
            jax.ShapeDtypeStruct((H, NP), jnp.float32),
            jax.ShapeDtypeStruct((NP, HD), jnp.float32),
        ],
    )(xpad, W1, att_src1, att_dst1, Wr1, br1r)


def _tc_b_body(acc_ref, den_ref, xr1_ref, b1_ref, w2_ref, as2_ref, ad2_ref,
               wr2_ref, br2_ref, xw2_ref, aux2_ref, xr2_ref):
    w2 = w2_ref[...]
    wr2 = wr2_ref[...]
    b1 = b1_ref[...]
    xw2 = jnp.zeros(xw2_ref.shape, jnp.float32)
    xr2 = jnp.zeros(xr2_ref.shape, jnp.float32)
    for h in range(H):
        d = jnp.maximum(den_ref[h], 1e-30)[:, None]
        hh = acc_ref[h] / d + b1[0, h * D:(h + 1) * D][None, :]
        hh = jnp.where(hh > 0, hh, jnp.exp(jnp.minimum(hh, 0.0)) - 1.0)
        hb = hh + xr1_ref[:, h * D:(h + 1) * D]
        xw2 = xw2 + _dot(hb, w2[h * D:(h + 1) * D, :])
        xr2 = xr2 + _dot(hb, wr2[h * D:(h + 1) * D, :])
    xw2_ref[...] = xw2
    aux2_ref[0] = jnp.sum(xw2 * as2_ref[...][0][None, :], axis=-1)
    aux2_ref[1] = jnp.sum(xw2 * ad2_ref[...][0][None, :], axis=-1)
    xr2_ref[...] = xr2 + br2_ref[...]


def _tc_b(acc1, den1, xr1, b1r, W2, att_src2, att_dst2, Wr2, br2r):
    B = 512
    G = NP // B
    return pl.pallas_call(
        _tc_b_body,
        grid=(G,),
        in_specs=[
            pl.BlockSpec((H, B, D), lambda i: (0, i, 0)),
            pl.BlockSpec((H, B), lambda i: (0, i)),
            pl.BlockSpec((B, HD), lambda i: (i, 0)),
            pl.BlockSpec((1, HD), lambda i: (0, 0)),
            pl.BlockSpec((HD, D), lambda i: (0, 0)),
            pl.BlockSpec((1, D), lambda i: (0, 0)),
            pl.BlockSpec((1, D), lambda i: (0, 0)),
            pl.BlockSpec((HD, D), lambda i: (0, 0)),
            pl.BlockSpec((1, D), lambda i: (0, 0)),
        ],
        out_specs=[
            pl.BlockSpec((B, D), lambda i: (i, 0)),
            pl.BlockSpec((2, B), lambda i: (0, i)),
            pl.BlockSpec((B, D), lambda i: (i, 0)),
        ],
        out_shape=[
            jax.ShapeDtypeStruct((NP, D), jnp.float32),
            jax.ShapeDtypeStruct((2, NP), jnp.float32),
            jax.ShapeDtypeStruct((NP, D), jnp.float32),
        ],
    )(acc1, den1, xr1, b1r, W2, att_src2, att_dst2, Wr2, br2r)


def _tc_c_body(acc_ref, den_ref, xr2_ref, b2_ref, out_ref):
    a = acc_ref[0] + acc_ref[1]
    d = jnp.maximum(den_ref[0] + den_ref[1], 1e-30)[:, None]
    h2 = a / d + b2_ref[...] + xr2_ref[...]
    m = jnp.max(h2, axis=1, keepdims=True)
    ex = jnp.exp(h2 - m)
    out_ref[...] = h2 - m - jnp.log(jnp.sum(ex, axis=1, keepdims=True))


def _tc_c(acc2, den2, xr2, b2r):
    B = 512
    G = NP // B
    return pl.pallas_call(
        _tc_c_body,
        grid=(G,),
        in_specs=[
            pl.BlockSpec((2, B, D), lambda i: (0, i, 0)),
            pl.BlockSpec((2, B), lambda i: (0, i)),
            pl.BlockSpec((B, D), lambda i: (i, 0)),
            pl.BlockSpec((1, D), lambda i: (0, 0)),
        ],
        out_specs=pl.BlockSpec((B, D), lambda i: (i, 0)),
        out_shape=jax.ShapeDtypeStruct((NP, D), jnp.float32),
    )(acc2, den2, xr2, b2r)


def _zero_fill(zrow, zden):
    z16 = jnp.zeros((16,), jnp.float32)
    for r in range(64):
        for q in range(D // 16):
            zrow[r, pl.ds(q * 16, 16)] = z16
    for r in range(RPT // 16):
        zden[pl.ds(r * 16, 16)] = z16


def _edge_sweep(nchunks, ebase, srcp, dstp, as_t, ad_t, idx_s, idx_d, exb,
                rowb, spm_tab, spm_acc, spm_den, sem0, sem1, sem2):
    def chunk(c, carry):
        base = ebase + c * CH
        pltpu.sync_copy(srcp.at[pl.ds(base, CH)], idx_s)
        pltpu.sync_copy(dstp.at[pl.ds(base, CH)], idx_d)
        g = pltpu.async_copy(spm_tab.at[idx_s], rowb, sem0)
        for j in range(CH // 16):
            sl = pl.ds(j * 16, 16)
            s16 = idx_s[sl]
            d16 = idx_d[sl]
            al = plsc.load_gather(as_t, [s16]) + plsc.load_gather(ad_t, [d16])
            al = jnp.where(al >= 0, al, al * jnp.float32(0.2))
            exb[sl] = jnp.exp(al)
        g.wait()
        for j in range(CH // 16):
            exv = exb[pl.ds(j * 16, 16)]
            for t in range(16):
                k = j * 16 + t
                ev = exv[t]
                for q in range(D // 16):
                    rowb[k, pl.ds(q * 16, 16)] = (
                        rowb[k, pl.ds(q * 16, 16)] * ev)
        pltpu.async_copy(rowb, spm_acc.at[idx_d], sem1, add=True).wait()
        pltpu.async_copy(exb, spm_den.at[idx_d], sem2, add=True).wait()
        return carry

    lax.fori_loop(0, nchunks, chunk, 0)


def _stage_and_zero(rb, tab_src, as_src, ad_src, as_t, ad_t,
                    spm_tab, spm_acc, spm_den, zrow, zden):
    pltpu.sync_copy(tab_src, spm_tab.at[pl.ds(rb, RPT)])
    for b in range(RPT // 64):
        pltpu.sync_copy(zrow, spm_acc.at[pl.ds(rb + b * 64, 64)])
    pltpu.sync_copy(zden, spm_den.at[pl.ds(rb, RPT)])
    pltpu.sync_copy(as_src, as_t)
    pltpu.sync_copy(ad_src, ad_t)


@functools.lru_cache(maxsize=None)
def _sc_kernels():
    scratch = [
        pltpu.VMEM_SHARED((NP, D), jnp.float32),
        pltpu.VMEM_SHARED((NP, D), jnp.float32),
        pltpu.VMEM_SHARED((NP,), jnp.float32),
        pltpu.VMEM((NP,), jnp.float32),
        pltpu.VMEM((NP,), jnp.float32),
        pltpu.VMEM((CH,), jnp.int32),
        pltpu.VMEM((CH,), jnp.int32),
        pltpu.VMEM((CH,), jnp.float32),
        pltpu.VMEM((CH, D), jnp.float32),
        pltpu.VMEM((64, D), jnp.float32),
        pltpu.VMEM((RPT,), jnp.float32),
        pltpu.SemaphoreType.DMA,
        pltpu.SemaphoreType.DMA,
        pltpu.SemaphoreType.DMA,
    ]
    mesh = plsc.VectorSubcoreMesh(core_axis_name="c", subcore_axis_name="s",
                                  num_cores=NC, num_subcores=NS)

    @functools.partial(
        pl.kernel,
        out_type=[
            jax.ShapeDtypeStruct((H, NP, D), jnp.float32),
            jax.ShapeDtypeStruct((H, NP), jnp.float32),
        ],
        mesh=mesh,
        scratch_types=scratch,
        compiler_params=pltpu.CompilerParams(needs_layout_passes=False),
    )
    def sc_l1(xw1h, a_s1, a_d1, srcp, dstp, acc1, den1,
              spm_tab, spm_acc, spm_den, as_t, ad_t, idx_s, idx_d, exb, rowb,
              zrow, zden, sem0, sem1, sem2):
        cid = lax.axis_index("c")
        tile = lax.axis_index("s")
        rb = tile * RPT
        _zero_fill(zrow, zden)
        epw = EP // NS
        nch = epw // CH
        for hi in range(H // NC):
            h = cid * (H // NC) + hi
            _stage_and_zero(rb, xw1h.at[h, pl.ds(rb, RPT)], a_s1.at[h],
                            a_d1.at[h], as_t, ad_t, spm_tab, spm_acc,
                            spm_den, zrow, zden)
            plsc.subcore_barrier()
            _edge_sweep(nch, tile * epw, srcp, dstp, as_t, ad_t, idx_s,
                        idx_d, exb, rowb, spm_tab, spm_acc, spm_den,
                        sem0, sem1, sem2)
            plsc.subcore_barrier()
            pltpu.sync_copy(spm_acc.at[pl.ds(rb, RPT)],
                            acc1.at[h, pl.ds(rb, RPT)])
            pltpu.sync_copy(spm_den.at[pl.ds(rb, RPT)],
                            den1.at[h, pl.ds(rb, RPT)])
            plsc.subcore_barrier()

    @functools.partial(
        pl.kernel,
        out_type=[
            jax.ShapeDtypeStruct((NC, NP, D), jnp.float32),
            jax.ShapeDtypeStruct((NC, NP), jnp.float32),
        ],
        mesh=mesh,
        scratch_types=scratch,
        compiler_params=pltpu.CompilerParams(needs_layout_passes=False),
    )
    def sc_l2(xw2, aux2, srcp, dstp, acc2, den2,
              spm_tab, spm_acc, spm_den, as_t, ad_t, idx_s, idx_d, exb, rowb,
              zrow, zden, sem0, sem1, sem2):
        cid = lax.axis_index("c")
        tile = lax.axis_index("s")
        rb = tile * RPT
        _zero_fill(zrow, zden)
        _stage_and_zero(rb, xw2.at[pl.ds(rb, RPT)], aux2.at[0], aux2.at[1],
                        as_t, ad_t, spm_tab, spm_acc, spm_den, zrow, zden)
        plsc.subcore_barrier()
        epc = EP // NC
        epw = epc // NS
        _edge_sweep(epw // CH, cid * epc + tile * epw, srcp, dstp, as_t,
                    ad_t, idx_s, idx_d, exb, rowb, spm_tab, spm_acc,
                    spm_den, sem0, sem1, sem2)
        plsc.subcore_barrier()
        pltpu.sync_copy(spm_acc.at[pl.ds(rb, RPT)],
                        acc2.at[cid, pl.ds(rb, RPT)])
        pltpu.sync_copy(spm_den.at[pl.ds(rb, RPT)],
                        den2.at[cid, pl.ds(rb, RPT)])

    return sc_l1, sc_l2


def kernel(x, edge_index, W1, att_src1, att_dst1, b1, W2, att_src2, att_dst2,
           b2, Wr1, br1, Wr2, br2):
    xpad = jnp.pad(x, ((0, NP - N), (0, 0)))
    loops = jnp.arange(N, dtype=jnp.int32)
    padv = (N + (jnp.arange(EP - ET, dtype=jnp.int32) % PADR)).astype(jnp.int32)
    srcp = jnp.concatenate([edge_index[0].astype(jnp.int32), loops, padv])
    dstp = jnp.concatenate([edge_index[1].astype(jnp.int32), loops, padv])

    sc_l1, sc_l2 = _sc_kernels()
    xw1h, a_s1, a_d1, xr1 = _tc_a(xpad, W1, att_src1, att_dst1, Wr1,
                                  br1.reshape(1, HD))
    acc1, den1 = sc_l1(xw1h, a_s1, a_d1, srcp, dstp)
    xw2, aux2, xr2 = _tc_b(acc1, den1, xr1, b1.reshape(1, HD), W2,
                           att_src2, att_dst2, Wr2, br2.reshape(1, D))
    acc2, den2 = sc_l2(xw2, aux2, srcp, dstp)
    out = _tc_c(acc2, den2, xr2, b2.reshape(1, D))
    return out[:N]


# speedup vs baseline: 1.2982x; 1.0410x over previous
import functools

import jax
import jax.numpy as jnp
from jax import lax
from jax.experimental import pallas as pl
from jax.experimental.pallas import tpu as pltpu
from jax.experimental.pallas import tpu_sc as plsc

N = 10000
NP = 10240
F = 128
D = 64
H = 8
HD = H * D
E = 320000
ET = E + N
NC = 2
NS = 16
CH = 128
EP = 331776
RPT = NP // NS
PADR = 8

_HIGH = jax.lax.Precision.HIGHEST


def _dot(a, b):
    return jnp.dot(a, b, precision=_HIGH, preferred_element_type=jnp.float32)


def _tc_a_body(x_ref, w1_ref, as_ref, ad_ref, wr1_ref, br1_ref,
               xw1h_ref, a_s1_ref, a_d1_ref, xr1_ref):
    xb = x_ref[...]
    w1 = w1_ref[...]
    a_s = as_ref[...]
    a_d = ad_ref[...]
    for h in range(H):
        xw_h = _dot(xb, w1[:, h * D:(h + 1) * D])
        xw1h_ref[h] = xw_h
        a_s1_ref[h] = jnp.sum(xw_h * a_s[h][None, :], axis=-1)
        a_d1_ref[h] = jnp.sum(xw_h * a_d[h][None, :], axis=-1)
    xr1_ref[...] = _dot(xb, wr1_ref[...]) + br1_ref[...]


def _tc_a(xpad, W1, att_src1, att_dst1, Wr1, br1r):
    B = 512
    G = NP // B
    return pl.pallas_call(
        _tc_a_body,
        grid=(G,),
        in_specs=[
            pl.BlockSpec((B, F), lambda i: (i, 0)),
            pl.BlockSpec((F, HD), lambda i: (0, 0)),
            pl.BlockSpec((H, D), lambda i: (0, 0)),
            pl.BlockSpec((H, D), lambda i: (0, 0)),
            pl.BlockSpec((F, HD), lambda i: (0, 0)),
            pl.BlockSpec((1, HD), lambda i: (0, 0)),
        ],
        out_specs=[
            pl.BlockSpec((H, B, D), lambda i: (0, i, 0)),
            pl.BlockSpec((H, B), lambda i: (0, i)),
            pl.BlockSpec((H, B), lambda i: (0, i)),
            pl.BlockSpec((B, HD), lambda i: (i, 0)),
        ],
        out_shape=[
            jax.ShapeDtypeStruct((H, NP, D), jnp.float32),
            jax.ShapeDtypeStruct((H, NP), jnp.float32),
            jax.ShapeDtypeStruct((H, NP), jnp.float32),
            jax.ShapeDtypeStruct((NP, HD), jnp.float32),
        ],
    )(xpad, W1, att_src1, att_dst1, Wr1, br1r)


def _tc_b_body(acc_ref, den_ref, xr1_ref, b1_ref, w2_ref, as2_ref, ad2_ref,
               wr2_ref, br2_ref, xw2_ref, aux2_ref, xr2_ref):
    w2 = w2_ref[...]
    wr2 = wr2_ref[...]
    b1 = b1_ref[...]
    xw2 = jnp.zeros(xw2_ref.shape, jnp.float32)
    xr2 = jnp.zeros(xr2_ref.shape, jnp.float32)
    for h in range(H):
        d = jnp.maximum(den_ref[h], 1e-30)[:, None]
        hh = acc_ref[h] / d + b1[0, h * D:(h + 1) * D][None, :]
        hh = jnp.where(hh > 0, hh, jnp.exp(jnp.minimum(hh, 0.0)) - 1.0)
        hb = hh + xr1_ref[:, h * D:(h + 1) * D]
        xw2 = xw2 + _dot(hb, w2[h * D:(h + 1) * D, :])
        xr2 = xr2 + _dot(hb, wr2[h * D:(h + 1) * D, :])
    xw2_ref[...] = xw2
    aux2_ref[0] = jnp.sum(xw2 * as2_ref[...][0][None, :], axis=-1)
    aux2_ref[1] = jnp.sum(xw2 * ad2_ref[...][0][None, :], axis=-1)
    xr2_ref[...] = xr2 + br2_ref[...]


def _tc_b(acc1, den1, xr1, b1r, W2, att_src2, att_dst2, Wr2, br2r):
    B = 512
    G = NP // B
    return pl.pallas_call(
        _tc_b_body,
        grid=(G,),
        in_specs=[
            pl.BlockSpec((H, B, D), lambda i: (0, i, 0)),
            pl.BlockSpec((H, B), lambda i: (0, i)),
            pl.BlockSpec((B, HD), lambda i: (i, 0)),
            pl.BlockSpec((1, HD), lambda i: (0, 0)),
            pl.BlockSpec((HD, D), lambda i: (0, 0)),
            pl.BlockSpec((1, D), lambda i: (0, 0)),
            pl.BlockSpec((1, D), lambda i: (0, 0)),
            pl.BlockSpec((HD, D), lambda i: (0, 0)),
            pl.BlockSpec((1, D), lambda i: (0, 0)),
        ],
        out_specs=[
            pl.BlockSpec((B, D), lambda i: (i, 0)),
            pl.BlockSpec((2, B), lambda i: (0, i)),
            pl.BlockSpec((B, D), lambda i: (i, 0)),
        ],
        out_shape=[
            jax.ShapeDtypeStruct((NP, D), jnp.float32),
            jax.ShapeDtypeStruct((2, NP), jnp.float32),
            jax.ShapeDtypeStruct((NP, D), jnp.float32),
        ],
    )(acc1, den1, xr1, b1r, W2, att_src2, att_dst2, Wr2, br2r)


def _tc_c_body(acc_ref, den_ref, xr2_ref, b2_ref, out_ref):
    a = acc_ref[0] + acc_ref[1]
    d = jnp.maximum(den_ref[0] + den_ref[1], 1e-30)[:, None]
    h2 = a / d + b2_ref[...] + xr2_ref[...]
    m = jnp.max(h2, axis=1, keepdims=True)
    ex = jnp.exp(h2 - m)
    out_ref[...] = h2 - m - jnp.log(jnp.sum(ex, axis=1, keepdims=True))


def _tc_c(acc2, den2, xr2, b2r):
    B = 512
    G = NP // B
    return pl.pallas_call(
        _tc_c_body,
        grid=(G,),
        in_specs=[
            pl.BlockSpec((2, B, D), lambda i: (0, i, 0)),
            pl.BlockSpec((2, B), lambda i: (0, i)),
            pl.BlockSpec((B, D), lambda i: (i, 0)),
            pl.BlockSpec((1, D), lambda i: (0, 0)),
        ],
        out_specs=pl.BlockSpec((B, D), lambda i: (i, 0)),
        out_shape=jax.ShapeDtypeStruct((NP, D), jnp.float32),
    )(acc2, den2, xr2, b2r)


def _zero_fill(zrow, zden):
    z16 = jnp.zeros((16,), jnp.float32)
    for r in range(64):
        for q in range(D // 16):
            zrow[r, pl.ds(q * 16, 16)] = z16
    for r in range(RPT // 16):
        zden[pl.ds(r * 16, 16)] = z16


def _edge_sweep(nchunks, ebase, srcp, dstp, as_t, ad_t, idx_s, idx_d, exb,
                rowb, spm_tab, spm_acc, spm_den, sem0, sem1, sem2):
    def chunk(c, carry):
        base = ebase + c * CH
        pltpu.sync_copy(srcp.at[pl.ds(base, CH)], idx_s)
        pltpu.sync_copy(dstp.at[pl.ds(base, CH)], idx_d)
        g = pltpu.async_copy(spm_tab.at[idx_s], rowb, sem0)
        for j in range(CH // 16):
            sl = pl.ds(j * 16, 16)
            s16 = idx_s[sl]
            d16 = idx_d[sl]
            al = plsc.load_gather(as_t, [s16]) + plsc.load_gather(ad_t, [d16])
            al = jnp.where(al >= 0, al, al * jnp.float32(0.2))
            exb[sl] = jnp.exp(al)
        dn = pltpu.async_copy(exb, spm_den.at[idx_d], sem2, add=True)
        g.wait()
        for j in range(CH // 16):
            exv = exb[pl.ds(j * 16, 16)]
            for t in range(16):
                k = j * 16 + t
                ev = exv[t]
                for q in range(D // 16):
                    rowb[k, pl.ds(q * 16, 16)] = (
                        rowb[k, pl.ds(q * 16, 16)] * ev)
        pltpu.async_copy(rowb, spm_acc.at[idx_d], sem1, add=True).wait()
        dn.wait()
        return carry

    lax.fori_loop(0, nchunks, chunk, 0)


def _stage_and_zero(rb, tab_src, as_src, ad_src, as_t, ad_t,
                    spm_tab, spm_acc, spm_den, zrow, zden):
    pltpu.sync_copy(tab_src, spm_tab.at[pl.ds(rb, RPT)])
    for b in range(RPT // 64):
        pltpu.sync_copy(zrow, spm_acc.at[pl.ds(rb + b * 64, 64)])
    pltpu.sync_copy(zden, spm_den.at[pl.ds(rb, RPT)])
    pltpu.sync_copy(as_src, as_t)
    pltpu.sync_copy(ad_src, ad_t)


@functools.lru_cache(maxsize=None)
def _sc_kernels():
    scratch = [
        pltpu.VMEM_SHARED((NP, D), jnp.float32),
        pltpu.VMEM_SHARED((NP, D), jnp.float32),
        pltpu.VMEM_SHARED((NP,), jnp.float32),
        pltpu.VMEM((NP,), jnp.float32),
        pltpu.VMEM((NP,), jnp.float32),
        pltpu.VMEM((CH,), jnp.int32),
        pltpu.VMEM((CH,), jnp.int32),
        pltpu.VMEM((CH,), jnp.float32),
        pltpu.VMEM((CH, D), jnp.float32),
        pltpu.VMEM((64, D), jnp.float32),
        pltpu.VMEM((RPT,), jnp.float32),
        pltpu.SemaphoreType.DMA,
        pltpu.SemaphoreType.DMA,
        pltpu.SemaphoreType.DMA,
    ]
    mesh = plsc.VectorSubcoreMesh(core_axis_name="c", subcore_axis_name="s",
                                  num_cores=NC, num_subcores=NS)

    @functools.partial(
        pl.kernel,
        out_type=[
            jax.ShapeDtypeStruct((H, NP, D), jnp.float32),
            jax.ShapeDtypeStruct((H, NP), jnp.float32),
        ],
        mesh=mesh,
        scratch_types=scratch,
        compiler_params=pltpu.CompilerParams(needs_layout_passes=False),
    )
    def sc_l1(xw1h, a_s1, a_d1, srcp, dstp, acc1, den1,
              spm_tab, spm_acc, spm_den, as_t, ad_t, idx_s, idx_d, exb, rowb,
              zrow, zden, sem0, sem1, sem2):
        cid = lax.axis_index("c")
        tile = lax.axis_index("s")
        rb = tile * RPT
        _zero_fill(zrow, zden)
        epw = EP // NS
        nch = epw // CH
        for hi in range(H // NC):
            h = cid * (H // NC) + hi
            _stage_and_zero(rb, xw1h.at[h, pl.ds(rb, RPT)], a_s1.at[h],
                            a_d1.at[h], as_t, ad_t, spm_tab, spm_acc,
                            spm_den, zrow, zden)
            plsc.subcore_barrier()
            _edge_sweep(nch, tile * epw, srcp, dstp, as_t, ad_t, idx_s,
                        idx_d, exb, rowb, spm_tab, spm_acc, spm_den,
                        sem0, sem1, sem2)
            plsc.subcore_barrier()
            pltpu.sync_copy(spm_acc.at[pl.ds(rb, RPT)],
                            acc1.at[h, pl.ds(rb, RPT)])
            pltpu.sync_copy(spm_den.at[pl.ds(rb, RPT)],
                            den1.at[h, pl.ds(rb, RPT)])
            plsc.subcore_barrier()

    @functools.partial(
        pl.kernel,
        out_type=[
            jax.ShapeDtypeStruct((NC, NP, D), jnp.float32),
            jax.ShapeDtypeStruct((NC, NP), jnp.float32),
        ],
        mesh=mesh,
        scratch_types=scratch,
        compiler_params=pltpu.CompilerParams(needs_layout_passes=False),
    )
    def sc_l2(xw2, aux2, srcp, dstp, acc2, den2,
              spm_tab, spm_acc, spm_den, as_t, ad_t, idx_s, idx_d, exb, rowb,
              zrow, zden, sem0, sem1, sem2):
        cid = lax.axis_index("c")
        tile = lax.axis_index("s")
        rb = tile * RPT
        _zero_fill(zrow, zden)
        _stage_and_zero(rb, xw2.at[pl.ds(rb, RPT)], aux2.at[0], aux2.at[1],
                        as_t, ad_t, spm_tab, spm_acc, spm_den, zrow, zden)
        plsc.subcore_barrier()
        epc = EP // NC
        epw = epc // NS
        _edge_sweep(epw // CH, cid * epc + tile * epw, srcp, dstp, as_t,
                    ad_t, idx_s, idx_d, exb, rowb, spm_tab, spm_acc,
                    spm_den, sem0, sem1, sem2)
        plsc.subcore_barrier()
        pltpu.sync_copy(spm_acc.at[pl.ds(rb, RPT)],
                        acc2.at[cid, pl.ds(rb, RPT)])
        pltpu.sync_copy(spm_den.at[pl.ds(rb, RPT)],
                        den2.at[cid, pl.ds(rb, RPT)])

    return sc_l1, sc_l2


def kernel(x, edge_index, W1, att_src1, att_dst1, b1, W2, att_src2, att_dst2,
           b2, Wr1, br1, Wr2, br2):
    xpad = jnp.pad(x, ((0, NP - N), (0, 0)))
    loops = jnp.arange(N, dtype=jnp.int32)
    padv = (N + (jnp.arange(EP - ET, dtype=jnp.int32) % PADR)).astype(jnp.int32)
    srcp = jnp.concatenate([edge_index[0].astype(jnp.int32), loops, padv])
    dstp = jnp.concatenate([edge_index[1].astype(jnp.int32), loops, padv])

    sc_l1, sc_l2 = _sc_kernels()
    xw1h, a_s1, a_d1, xr1 = _tc_a(xpad, W1, att_src1, att_dst1, Wr1,
                                  br1.reshape(1, HD))
    acc1, den1 = sc_l1(xw1h, a_s1, a_d1, srcp, dstp)
    xw2, aux2, xr2 = _tc_b(acc1, den1, xr1, b1.reshape(1, HD), W2,
                           att_src2, att_dst2, Wr2, br2.reshape(1, D))
    acc2, den2 = sc_l2(xw2, aux2, srcp, dstp)
    out = _tc_c(acc2, den2, xr2, b2.reshape(1, D))
    return out[:N]
